# baseline (speedup 1.0000x reference)
import functools

import jax
import jax.numpy as jnp
from jax import lax
from jax.experimental import pallas as pl
from jax.experimental.pallas import tpu as pltpu
from jax.experimental.pallas import tpu_sc as plsc

BATCH_N = 16384
DIM = 128

_info = plsc.get_sparse_core_info()
_NC = _info.num_cores
_NS = _info.num_subcores
_NW = _NC * _NS
_BPW = BATCH_N // _NW
_CHUNK = 128
_NCH = _BPW // _CHUNK

_mesh = plsc.VectorSubcoreMesh(core_axis_name="c", subcore_axis_name="s")


@functools.partial(
    pl.kernel,
    out_type=(
        jax.ShapeDtypeStruct((BATCH_N, DIM), jnp.float32),
        jax.ShapeDtypeStruct((BATCH_N, DIM), jnp.float32),
    ),
    mesh=_mesh,
    scratch_types=[
        pltpu.VMEM((_NCH, _CHUNK), jnp.int32),
        pltpu.VMEM((4, _CHUNK, DIM), jnp.float32),
        pltpu.SemaphoreType.DMA,
        pltpu.SemaphoreType.DMA,
        pltpu.SemaphoreType.DMA,
        pltpu.SemaphoreType.DMA,
        pltpu.SemaphoreType.DMA,
        pltpu.SemaphoreType.DMA,
        pltpu.SemaphoreType.DMA,
        pltpu.SemaphoreType.DMA,
    ],
)
def _dual_gather(ids_hbm, shape_hbm, tex_hbm, out_s, out_t,
                 idx_v, bufs_v,
                 gi0, gi1, gi2, gi3, go0, go1, go2, go3):
    _NBUF = 4
    wid = lax.axis_index("s") * _NC + lax.axis_index("c")
    base = wid * _BPW
    pltpu.sync_copy(ids_hbm.at[pl.ds(wid * _NCH, _NCH)], idx_v)

    sem_in = (gi0, gi1, gi2, gi3)
    sem_out = (go0, go1, go2, go3)
    tabs = (shape_hbm, tex_hbm)
    outs = (out_s, out_t)
    jobs = [(t, j) for j in range(_NCH) for t in range(2)]
    njobs = len(jobs)

    in_h = [None] * _NBUF
    out_h = [None] * _NBUF

    def drain(k):
        bk = k % _NBUF
        tk, jk = jobs[k]
        in_h[bk].wait()
        out_h[bk] = pltpu.async_copy(
            bufs_v.at[bk],
            outs[tk].at[pl.ds(base + jk * _CHUNK, _CHUNK)],
            sem_out[bk])

    for i, (t, j) in enumerate(jobs):
        b = i % _NBUF
        if out_h[b] is not None:
            out_h[b].wait()
            out_h[b] = None
        in_h[b] = pltpu.async_copy(
            tabs[t].at[idx_v.at[j]], bufs_v.at[b], sem_in[b])
        k = i - (_NBUF - 1)
        if k >= 0:
            drain(k)
    for k in range(max(0, njobs - (_NBUF - 1)), njobs):
        drain(k)
    for b in range(_NBUF):
        if out_h[b] is not None:
            out_h[b].wait()


def kernel(object_ids, shape_code, texture_code):
    ids2d = object_ids.astype(jnp.int32).reshape(_NW * _NCH, _CHUNK)
    return _dual_gather(ids2d, shape_code, texture_code)


# speedup vs baseline: 1.0055x; 1.0055x over previous
import functools

import jax
import jax.numpy as jnp
from jax import lax
from jax.experimental import pallas as pl
from jax.experimental.pallas import tpu as pltpu
from jax.experimental.pallas import tpu_sc as plsc

BATCH_N = 16384
DIM = 128

_info = plsc.get_sparse_core_info()
_NC = _info.num_cores
_NS = _info.num_subcores
_NW = _NC * _NS
_BPW = BATCH_N // _NW
_CHUNK = 128
_NCH = _BPW // _CHUNK

_mesh = plsc.VectorSubcoreMesh(core_axis_name="c", subcore_axis_name="s")


@functools.partial(
    pl.kernel,
    out_type=(
        jax.ShapeDtypeStruct((BATCH_N, DIM), jnp.float32),
        jax.ShapeDtypeStruct((BATCH_N, DIM), jnp.float32),
    ),
    mesh=_mesh,
    scratch_types=[
        pltpu.VMEM((_NCH, _CHUNK), jnp.int32),
        pltpu.VMEM((4, _CHUNK, DIM), jnp.float32),
        pltpu.SemaphoreType.DMA,
        pltpu.SemaphoreType.DMA,
        pltpu.SemaphoreType.DMA,
        pltpu.SemaphoreType.DMA,
        pltpu.SemaphoreType.DMA,
        pltpu.SemaphoreType.DMA,
        pltpu.SemaphoreType.DMA,
        pltpu.SemaphoreType.DMA,
    ],
)
def _dual_gather(ids_hbm, shape_hbm, tex_hbm, out_s, out_t,
                 idx_v, bufs_v,
                 gi0, gi1, gi2, gi3, go0, go1, go2, go3):
    _NBUF = 4
    wid = lax.axis_index("s") * _NC + lax.axis_index("c")
    base = wid * _BPW
    pltpu.sync_copy(ids_hbm.at[pl.ds(wid * _NCH, _NCH)], idx_v)

    sem_in = (gi0, gi1, gi2, gi3)
    sem_out = (go0, go1, go2, go3)
    tabs = (shape_hbm, tex_hbm)
    outs = (out_s, out_t)
    jobs = [(t, j) for t in range(2) for j in range(_NCH)]
    njobs = len(jobs)

    in_h = [None] * _NBUF
    out_h = [None] * _NBUF

    def drain(k):
        bk = k % _NBUF
        tk, jk = jobs[k]
        in_h[bk].wait()
        out_h[bk] = pltpu.async_copy(
            bufs_v.at[bk],
            outs[tk].at[pl.ds(base + jk * _CHUNK, _CHUNK)],
            sem_out[bk])

    for i, (t, j) in enumerate(jobs):
        b = i % _NBUF
        if out_h[b] is not None:
            out_h[b].wait()
            out_h[b] = None
        in_h[b] = pltpu.async_copy(
            tabs[t].at[idx_v.at[j]], bufs_v.at[b], sem_in[b])
        k = i - (_NBUF - 1)
        if k >= 0:
            drain(k)
    for k in range(max(0, njobs - (_NBUF - 1)), njobs):
        drain(k)
    for b in range(_NBUF):
        if out_h[b] is not None:
            out_h[b].wait()


def kernel(object_ids, shape_code, texture_code):
    ids2d = object_ids.astype(jnp.int32).reshape(_NW * _NCH, _CHUNK)
    return _dual_gather(ids2d, shape_code, texture_code)


# speedup vs baseline: 1.0149x; 1.0093x over previous
import functools

import jax
import jax.numpy as jnp
from jax import lax
from jax.experimental import pallas as pl
from jax.experimental.pallas import tpu as pltpu
from jax.experimental.pallas import tpu_sc as plsc

BATCH_N = 16384
DIM = 128

_info = plsc.get_sparse_core_info()
_NC = _info.num_cores
_NS = _info.num_subcores
_NW = _NC * _NS
_BPW = BATCH_N // _NW
_CHUNK = 128
_NCH = _BPW // _CHUNK
_HALF = _CHUNK // 2

_mesh = plsc.VectorSubcoreMesh(core_axis_name="c", subcore_axis_name="s")

_SCHED0 = [(0, 0, _HALF, 0), (0, _HALF, _HALF, _HALF),
           (1, 0, _CHUNK, _CHUNK), (2, 0, _CHUNK, 2 * _CHUNK),
           (3, 0, _CHUNK, 3 * _CHUNK)]
_SCHED1 = [(0, 0, _CHUNK, 0), (1, 0, _CHUNK, _CHUNK),
           (2, 0, _CHUNK, 2 * _CHUNK),
           (3, 0, _HALF, 3 * _CHUNK), (3, _HALF, _HALF, 3 * _CHUNK + _HALF)]


@functools.partial(
    pl.kernel,
    out_type=(
        jax.ShapeDtypeStruct((BATCH_N, DIM), jnp.float32),
        jax.ShapeDtypeStruct((BATCH_N, DIM), jnp.float32),
    ),
    mesh=_mesh,
    scratch_types=[
        pltpu.VMEM((_NCH, _CHUNK), jnp.int32),
        pltpu.VMEM((4, _CHUNK, DIM), jnp.float32),
        pltpu.SemaphoreType.DMA,
        pltpu.SemaphoreType.DMA,
        pltpu.SemaphoreType.DMA,
        pltpu.SemaphoreType.DMA,
        pltpu.SemaphoreType.DMA,
        pltpu.SemaphoreType.DMA,
        pltpu.SemaphoreType.DMA,
        pltpu.SemaphoreType.DMA,
    ],
)
def _dual_gather(ids_hbm, shape_hbm, tex_hbm, out_s, out_t,
                 idx_v, bufs_v,
                 gi0, gi1, gi2, gi3, go0, go1, go2, go3):
    _NBUF = 4
    wid = lax.axis_index("s") * _NC + lax.axis_index("c")
    base = wid * _BPW
    pltpu.sync_copy(ids_hbm.at[pl.ds(wid * _NCH, _NCH)], idx_v)

    sem_in = (gi0, gi1, gi2, gi3)
    sem_out = (go0, go1, go2, go3)
    jobs = ([(out_s, c) for c in _SCHED0] + [(out_t, c) for c in _SCHED1])
    tabs = [shape_hbm] * len(_SCHED0) + [tex_hbm] * len(_SCHED1)
    njobs = len(jobs)

    in_h = [None] * _NBUF
    out_h = [None] * _NBUF

    def drain(k):
        bk = k % _NBUF
        out_ref, (_, _, n, off) = jobs[k]
        in_h[bk].wait()
        out_h[bk] = pltpu.async_copy(
            bufs_v.at[bk, pl.ds(0, n)],
            out_ref.at[pl.ds(base + off, n)],
            sem_out[bk])

    for i in range(njobs):
        b = i % _NBUF
        _, (row, lo, n, _) = jobs[i]
        if out_h[b] is not None:
            out_h[b].wait()
            out_h[b] = None
        in_h[b] = pltpu.async_copy(
            tabs[i].at[idx_v.at[row, pl.ds(lo, n)]],
            bufs_v.at[b, pl.ds(0, n)], sem_in[b])
        k = i - (_NBUF - 1)
        if k >= 0:
            drain(k)
    for k in range(max(0, njobs - (_NBUF - 1)), njobs):
        drain(k)
    for b in range(_NBUF):
        if out_h[b] is not None:
            out_h[b].wait()


def kernel(object_ids, shape_code, texture_code):
    ids2d = object_ids.astype(jnp.int32).reshape(_NW * _NCH, _CHUNK)
    return _dual_gather(ids2d, shape_code, texture_code)
